# baseline (speedup 1.0000x reference)
import jax
import jax.numpy as jnp
from jax import lax
from jax.experimental import pallas as pl
from jax.experimental.pallas import tpu as pltpu
from jax.experimental.pallas import tpu_sc as plsc

N = 50000
E = 800000
H = 64
HH = 32
C = 128
NSUB = 16
EPT = 50176
EPAD = EPT * NSUB
NPAD = 50048
NPT = NPAD // NSUB
CPT = EPT // C
SCH = 1
NSC = CPT // SCH
ROWS = SCH * C
NCHK = EPAD // C
ZROWS = 391
BN = 2000
NB = N // BN
BE = 16000
PADC = float(EPAD - E)

_mesh = plsc.VectorSubcoreMesh(core_axis_name="c", subcore_axis_name="s")

_GDN = lax.GatherDimensionNumbers(offset_dims=(), collapsed_slice_dims=(0,),
                                  start_index_map=(0,))


def _vgather16(v, idx16):
    return lax.gather(v, idx16[:, None], _GDN, (1,),
                      indices_are_sorted=False, unique_indices=False,
                      mode=lax.GatherScatterMode.PROMISE_IN_BOUNDS)


def _deg_body(src2, dst2, degS, degD, ib0, ib1, ib2, ib3, onesv, zbuf, acc,
              is0, is1, is2, is3, ss0, ss1):
    c = lax.axis_index("c")
    s = lax.axis_index("s")
    ibufs = [ib0, ib1, ib2, ib3]
    isems = [is0, is1, is2, is3]
    ssems = [ss0, ss1]
    ones16 = jnp.ones((16,), jnp.float32)
    zero16 = jnp.zeros((16,), jnp.float32)

    def fill_ones(r, _):
        onesv[r, pl.ds(0, 16)] = ones16
        return _
    lax.fori_loop(0, C, fill_ones, 0)

    def fill_zero(r, _):
        zbuf[r, pl.ds(0, 16)] = zero16
        return _
    lax.fori_loop(0, ZROWS, fill_zero, 0)

    def zero_acc(i, _):
        pltpu.sync_copy(zbuf, acc.at[pl.ds(s * NPT + i * ZROWS, ZROWS)])
        return _
    lax.fori_loop(0, NPT // ZROWS, zero_acc, 0)
    plsc.subcore_barrier()

    base = s * CPT

    def src_rows(t):
        return src2.at[pl.ds(base + t * SCH, SCH)]

    def dst_rows(t):
        return dst2.at[pl.ds(base + t * SCH, SCH)]

    def fire_idx(t, b):
        @pl.when(c == 0)
        def _():
            pltpu.async_copy(src_rows(t), ibufs[b], isems[b])

        @pl.when(c == 1)
        def _():
            pltpu.async_copy(dst_rows(t), ibufs[b], isems[b])

    def wait_idx(t, b):
        @pl.when(c == 0)
        def _():
            pltpu.make_async_copy(src_rows(t), ibufs[b], isems[b]).wait()

        @pl.when(c == 1)
        def _():
            pltpu.make_async_copy(dst_rows(t), ibufs[b], isems[b]).wait()

    @pl.when(c == 0)
    def _():
        pltpu.sync_copy(src_rows(0), ib0)

    @pl.when(c == 1)
    def _():
        pltpu.sync_copy(dst_rows(0), ib0)

    fire_idx(1, 1)

    def step(i, carry):
        for P in range(4):
            t = 4 * i + P
            ib, ise, sse = ibufs[P], isems[P], ssems[P & 1]

            @pl.when(t > 0)
            def _():
                wait_idx(t, P)

            @pl.when(t >= 2)
            def _():
                for _k in range(SCH):
                    pltpu.make_async_copy(
                        onesv, acc.at[pl.ds(0, C)], sse).wait()

            for k in range(SCH):
                pltpu.async_copy(onesv, acc.at[ib.at[k]], sse, add=True)

            @pl.when(t < NSC - 2)
            def _():
                fire_idx(t + 2, (P + 2) % 4)
        return carry
    lax.fori_loop(0, NSC // 4, step, 0)

    for par in range(2):
        for _k in range(SCH):
            pltpu.make_async_copy(onesv, acc.at[pl.ds(0, C)],
                                  ssems[par]).wait()
    plsc.subcore_barrier()

    @pl.when(c == 0)
    def _():
        pltpu.sync_copy(acc.at[pl.ds(s * NPT, NPT)],
                        degS.at[pl.ds(s * NPT, NPT)])

    @pl.when(c == 1)
    def _():
        pltpu.sync_copy(acc.at[pl.ds(s * NPT, NPT)],
                        degD.at[pl.ds(s * NPT, NPT)])


@jax.jit
def _deg(src2, dst2):
    return pl.kernel(
        _deg_body,
        out_type=[jax.ShapeDtypeStruct((NPAD, 16), jnp.float32),
                  jax.ShapeDtypeStruct((NPAD, 16), jnp.float32)],
        mesh=_mesh,
        scratch_types=(
            [pltpu.VMEM((SCH, C), jnp.int32)] * 4
            + [pltpu.VMEM((C, 16), jnp.float32),
               pltpu.VMEM((ZROWS, 16), jnp.float32),
               pltpu.VMEM_SHARED((NPAD, 16), jnp.float32)]
            + [pltpu.SemaphoreType.DMA] * 6
        ),
        compiler_params=pltpu.CompilerParams(use_tc_tiling_on_sc=False),
    )(src2, dst2)


def _agg_body(x, src2, dst2, ew2, outA, outB,
              sb0, sb1, sb2, sb3, sb4, sb5, sb6, sb7,
              db0, db1, db2, db3, db4, db5, db6, db7,
              eb0, eb1, eb2, eb3, eb4, eb5, eb6, eb7,
              rows0, rows1, rows2, rows3, wrows0, wrows1, acc,
              is0, is1, is2, is3, is4, is5, is6, is7,
              gs0, gs1, gs2, gs3, ss0, ss1):
    c = lax.axis_index("c")
    s = lax.axis_index("s")
    sbufs = [sb0, sb1, sb2, sb3, sb4, sb5, sb6, sb7]
    dbufs = [db0, db1, db2, db3, db4, db5, db6, db7]
    ebufs = [eb0, eb1, eb2, eb3, eb4, eb5, eb6, eb7]
    rowsb = [rows0, rows1, rows2, rows3]
    wrowsb = [wrows0, wrows1]
    isems = [is0, is1, is2, is3, is4, is5, is6, is7]
    gsems = [gs0, gs1, gs2, gs3]
    ssems = [ss0, ss1]
    zero16 = jnp.zeros((16,), jnp.float32)

    def fill_zero(r, _):
        wrows0[r, pl.ds(0, 16)] = zero16
        wrows0[r, pl.ds(16, 16)] = zero16
        return _
    lax.fori_loop(0, ROWS, fill_zero, 0)

    def zero_acc(i, _):
        pltpu.sync_copy(wrows0, acc.at[pl.ds(s * NPT + i * ROWS, ROWS)])
        return _
    lax.fori_loop(0, 24, zero_acc, 0)
    pltpu.sync_copy(wrows0.at[pl.ds(0, 56)],
                    acc.at[pl.ds(s * NPT + 24 * ROWS, 56)])
    plsc.subcore_barrier()

    base = s * CPT
    off16 = jnp.zeros((16,), jnp.int32) + c * N

    def add_off(b):
        sb = sbufs[b]
        for g in range(8):
            sb[0, pl.ds(g * 16, 16)] = sb[0, pl.ds(g * 16, 16)] + off16

    def s_rows(t):
        return src2.at[pl.ds(base + t, 1)]

    def d_rows(t):
        return dst2.at[pl.ds(base + t, 1)]

    def e_rows(t):
        return ew2.at[pl.ds(base + t, 1)]

    def fire_idx(t, b):
        pltpu.async_copy(s_rows(t), sbufs[b], isems[b])
        pltpu.async_copy(d_rows(t), dbufs[b], isems[b])
        pltpu.async_copy(e_rows(t), ebufs[b], isems[b])

    def wait_idx(t, b):
        pltpu.make_async_copy(s_rows(t), sbufs[b], isems[b]).wait()
        pltpu.make_async_copy(d_rows(t), dbufs[b], isems[b]).wait()
        pltpu.make_async_copy(e_rows(t), ebufs[b], isems[b]).wait()

    pltpu.sync_copy(s_rows(0), sb0)
    pltpu.sync_copy(d_rows(0), db0)
    pltpu.sync_copy(e_rows(0), eb0)
    pltpu.sync_copy(s_rows(1), sb1)
    pltpu.sync_copy(d_rows(1), db1)
    pltpu.sync_copy(e_rows(1), eb1)
    fire_idx(2, 2)
    fire_idx(3, 3)
    add_off(0)
    add_off(1)
    pltpu.async_copy(x.at[sb0.at[0]], rows0, gs0)
    pltpu.async_copy(x.at[sb1.at[0]], rows1, gs1)

    def step(i, carry):
        for P in range(8):
            t = 8 * i + P
            P2 = (P + 2) % 8
            P4 = (P + 4) % 8
            r4 = P % 4
            g2 = (P + 2) % 4
            par = P & 1

            @pl.when(t < NSC - 2)
            def _():
                wait_idx(t + 2, P2)
                add_off(P2)
                pltpu.async_copy(x.at[sbufs[P2].at[0]], rowsb[g2],
                                 gsems[g2])

            pltpu.make_async_copy(x.at[pl.ds(0, C)], rowsb[r4],
                                  gsems[r4]).wait()

            @pl.when(t >= 2)
            def _():
                pltpu.make_async_copy(wrowsb[par], acc.at[pl.ds(0, C)],
                                      ssems[par]).wait()

            rb = rowsb[r4]
            wb = wrowsb[par]
            eb = ebufs[P]

            def wgrp(g, _2):
                w16 = eb[0, pl.ds(g * 16, 16)]
                rbase = g * 16
                for j in range(16):
                    r = rbase + j
                    w = _vgather16(w16, jnp.full((16,), j, jnp.int32))
                    lo = rb[r, pl.ds(0, 16)].astype(jnp.float32)
                    hi = rb[r, pl.ds(16, 16)].astype(jnp.float32)
                    wb[r, pl.ds(0, 16)] = lo * w
                    wb[r, pl.ds(16, 16)] = hi * w
                return _2
            lax.fori_loop(0, 8, wgrp, 0)

            pltpu.async_copy(wb, acc.at[dbufs[P].at[0]], ssems[par],
                             add=True)

            @pl.when(t < NSC - 4)
            def _():
                fire_idx(t + 4, P4)
        return carry
    lax.fori_loop(0, NSC // 8, step, 0)

    for par in range(2):
        pltpu.make_async_copy(wrowsb[par], acc.at[pl.ds(0, C)],
                              ssems[par]).wait()
    plsc.subcore_barrier()

    @pl.when(c == 0)
    def _():
        pltpu.sync_copy(acc.at[pl.ds(s * NPT, NPT)],
                        outA.at[pl.ds(s * NPT, NPT)])

    @pl.when(c == 1)
    def _():
        pltpu.sync_copy(acc.at[pl.ds(s * NPT, NPT)],
                        outB.at[pl.ds(s * NPT, NPT)])


@jax.jit
def _agg(x, src2, dst2, ew2):
    return pl.kernel(
        _agg_body,
        out_type=[jax.ShapeDtypeStruct((NPAD, HH), jnp.float32),
                  jax.ShapeDtypeStruct((NPAD, HH), jnp.float32)],
        mesh=_mesh,
        scratch_types=(
            [pltpu.VMEM((1, C), jnp.int32)] * 16
            + [pltpu.VMEM((1, C), jnp.float32)] * 8
            + [pltpu.VMEM((ROWS, HH), jnp.bfloat16)] * 4
            + [pltpu.VMEM((ROWS, HH), jnp.float32)] * 2
            + [pltpu.VMEM_SHARED((NPAD, HH), jnp.float32)]
            + [pltpu.SemaphoreType.DMA] * 14
        ),
        compiler_params=pltpu.CompilerParams(use_tc_tiling_on_sc=False),
    )(x, src2, dst2, ew2)


def _relu(v):
    return jnp.maximum(v, 0.0)


def _enc_edges_body(bond_ref, we_ref, be_ref, ew_ref):
    zt = lax.dot_general(we_ref[...], bond_ref[...],
                         (((0,), (1,)), ((), ())),
                         preferred_element_type=jnp.float32)
    e = _relu(zt + be_ref[...])
    ew_ref[...] = jnp.mean(e, axis=0, keepdims=True)


@jax.jit
def _enc_edges(bond, We, beT):
    return pl.pallas_call(
        _enc_edges_body,
        grid=(E // BE,),
        in_specs=[
            pl.BlockSpec((BE, 10), lambda i: (i, 0)),
            pl.BlockSpec((10, H), lambda i: (0, 0)),
            pl.BlockSpec((H, 1), lambda i: (0, 0)),
        ],
        out_specs=pl.BlockSpec((1, BE), lambda i: (0, i)),
        out_shape=jax.ShapeDtypeStruct((1, E), jnp.float32),
    )(bond, We, beT)


def _norm_from_deg(deg_col, blk_i):
    rowz = lax.broadcasted_iota(jnp.int32, (BN, 1), 0) == 0
    fix = jnp.where(rowz & (blk_i == 0), PADC, 0.0)
    d = jnp.maximum(deg_col - fix, 1.0)
    return lax.rsqrt(d)


def _enc_nodes_body(atom_ref, lap_ref, wa_ref, wb_ref, b_ref, deg_ref, x_ref):
    i = pl.program_id(0)
    c = pl.program_id(1)
    h = _relu(jnp.dot(atom_ref[...], wa_ref[...],
                      preferred_element_type=jnp.float32)
              + jnp.dot(lap_ref[...], wb_ref[...],
                        preferred_element_type=jnp.float32)
              + b_ref[...])
    x = h * _norm_from_deg(deg_ref[:, 0:1], i)
    x_ref[...] = jnp.where(c == 0, x[:, :HH], x[:, HH:]).astype(jnp.bfloat16)


@jax.jit
def _enc_nodes(atom, lap_pe, Wa, Wb, b2, degS):
    return pl.pallas_call(
        _enc_nodes_body,
        grid=(NB, 2),
        in_specs=[
            pl.BlockSpec((BN, 44), lambda i, c: (i, 0)),
            pl.BlockSpec((BN, 8), lambda i, c: (i, 0)),
            pl.BlockSpec((44, H), lambda i, c: (0, 0)),
            pl.BlockSpec((8, H), lambda i, c: (0, 0)),
            pl.BlockSpec((1, H), lambda i, c: (0, 0)),
            pl.BlockSpec((BN, 16), lambda i, c: (i, 0)),
        ],
        out_specs=pl.BlockSpec((BN, HH), lambda i, c: (c * NB + i, 0)),
        out_shape=jax.ShapeDtypeStruct((2 * N, HH), jnp.bfloat16),
    )(atom, lap_pe, Wa, Wb, b2, degS)


def _mm_body(aggA_ref, aggB_ref, deg_ref, w_ref, b_ref, y_ref, st_ref):
    i = pl.program_id(0)
    z = jnp.concatenate([aggA_ref[...], aggB_ref[...]], axis=1)
    z = z * _norm_from_deg(deg_ref[:, 0:1], i)
    y = jnp.dot(z, w_ref[...], preferred_element_type=jnp.float32) + b_ref[...]
    y_ref[...] = y
    s1 = jnp.sum(y, axis=0)
    s2 = jnp.sum(y * y, axis=0)
    z64 = jnp.zeros((1, H), jnp.float32)
    st = jnp.concatenate(
        [jnp.concatenate([s1.reshape(1, H), z64], axis=1),
         jnp.concatenate([s2.reshape(1, H), z64], axis=1),
         jnp.zeros((6, 128), jnp.float32)], axis=0)

    @pl.when(i == 0)
    def _():
        st_ref[...] = st

    @pl.when(i > 0)
    def _():
        st_ref[...] = st_ref[...] + st


@jax.jit
def _mm(aggA, aggB, degD, W, b2):
    return pl.pallas_call(
        _mm_body,
        grid=(NB,),
        in_specs=[
            pl.BlockSpec((BN, HH), lambda i: (i, 0)),
            pl.BlockSpec((BN, HH), lambda i: (i, 0)),
            pl.BlockSpec((BN, 16), lambda i: (i, 0)),
            pl.BlockSpec((H, H), lambda i: (0, 0)),
            pl.BlockSpec((1, H), lambda i: (0, 0)),
        ],
        out_specs=[
            pl.BlockSpec((BN, H), lambda i: (i, 0)),
            pl.BlockSpec((8, 128), lambda i: (0, 0)),
        ],
        out_shape=[
            jax.ShapeDtypeStruct((N, H), jnp.float32),
            jax.ShapeDtypeStruct((8, 128), jnp.float32),
        ],
    )(aggA, aggB, degD, W, b2)


def _bn(y, st_ref, g_ref, bt_ref):
    mu = st_ref[0:1, 0:H] / N
    var = st_ref[1:2, 0:H] / N - mu * mu
    return _relu(g_ref[...] * (y - mu) * lax.rsqrt(var + 1e-5) + bt_ref[...])


def _bn_scale_body(y_ref, st_ref, g_ref, bt_ref, deg_ref, x_ref):
    i = pl.program_id(0)
    c = pl.program_id(1)
    h = _bn(y_ref[...], st_ref, g_ref, bt_ref)
    x = h * _norm_from_deg(deg_ref[:, 0:1], i)
    x_ref[...] = jnp.where(c == 0, x[:, :HH], x[:, HH:]).astype(jnp.bfloat16)


@jax.jit
def _bn_scale(y, st, g2, bt2, degS):
    return pl.pallas_call(
        _bn_scale_body,
        grid=(NB, 2),
        in_specs=[
            pl.BlockSpec((BN, H), lambda i, c: (i, 0)),
            pl.BlockSpec((8, 128), lambda i, c: (0, 0)),
            pl.BlockSpec((1, H), lambda i, c: (0, 0)),
            pl.BlockSpec((1, H), lambda i, c: (0, 0)),
            pl.BlockSpec((BN, 16), lambda i, c: (i, 0)),
        ],
        out_specs=pl.BlockSpec((BN, HH), lambda i, c: (c * NB + i, 0)),
        out_shape=jax.ShapeDtypeStruct((2 * N, HH), jnp.bfloat16),
    )(y, st, g2, bt2, degS)


def _bn_h_body(y_ref, st_ref, g_ref, bt_ref, h_ref):
    h_ref[...] = _bn(y_ref[...], st_ref, g_ref, bt_ref)


@jax.jit
def _bn_h(y, st, g2, bt2):
    return pl.pallas_call(
        _bn_h_body,
        grid=(NB,),
        in_specs=[
            pl.BlockSpec((BN, H), lambda i: (i, 0)),
            pl.BlockSpec((8, 128), lambda i: (0, 0)),
            pl.BlockSpec((1, H), lambda i: (0, 0)),
            pl.BlockSpec((1, H), lambda i: (0, 0)),
        ],
        out_specs=pl.BlockSpec((BN, H), lambda i: (i, 0)),
        out_shape=jax.ShapeDtypeStruct((N, H), jnp.float32),
    )(y, st, g2, bt2)


def _gate_body(h_ref, wg_ref, bg_ref, gate_ref, gmax_ref):
    i = pl.program_id(0)
    gate = jnp.sum(h_ref[...] * wg_ref[...], axis=1, keepdims=True) + bg_ref[...]
    gate_ref[...] = gate
    m = jnp.max(gate).reshape(1, 1)

    @pl.when(i == 0)
    def _():
        gmax_ref[...] = m

    @pl.when(i > 0)
    def _():
        gmax_ref[...] = jnp.maximum(gmax_ref[...], m)


@jax.jit
def _gate(h, wg2, bg2):
    return pl.pallas_call(
        _gate_body,
        grid=(NB,),
        in_specs=[
            pl.BlockSpec((BN, H), lambda i: (i, 0)),
            pl.BlockSpec((1, H), lambda i: (0, 0)),
            pl.BlockSpec((1, 1), lambda i: (0, 0)),
        ],
        out_specs=[
            pl.BlockSpec((BN, 1), lambda i: (i, 0)),
            pl.BlockSpec((1, 1), lambda i: (0, 0)),
        ],
        out_shape=[
            jax.ShapeDtypeStruct((N, 1), jnp.float32),
            jax.ShapeDtypeStruct((1, 1), jnp.float32),
        ],
    )(h, wg2, bg2)


def _pool_body(h_ref, gate_ref, gmax_ref, wo1_ref, bo1_ref, wo2_ref, bo2_ref,
               out_ref, acc_ref):
    i = pl.program_id(0)
    a = jnp.exp(gate_ref[...] - gmax_ref[0, 0])
    s = jnp.sum(a)
    r = jnp.sum(a * h_ref[...], axis=0)
    cur = jnp.concatenate(
        [jnp.concatenate([r.reshape(1, H), jnp.zeros((1, H), jnp.float32)],
                         axis=1),
         jnp.concatenate([s.reshape(1, 1), jnp.zeros((1, 127), jnp.float32)],
                         axis=1),
         jnp.zeros((6, 128), jnp.float32)], axis=0)

    @pl.when(i == 0)
    def _():
        acc_ref[...] = cur

    @pl.when(i > 0)
    def _():
        acc_ref[...] = acc_ref[...] + cur

    @pl.when(i == NB - 1)
    def _():
        rd = acc_ref[0:1, 0:H] / acc_ref[1, 0]
        o = _relu(jnp.dot(rd, wo1_ref[...],
                          preferred_element_type=jnp.float32) + bo1_ref[...])
        out_ref[...] = jnp.dot(o, wo2_ref[...],
                               preferred_element_type=jnp.float32) + bo2_ref[...]


@jax.jit
def _pool(h, gate, gmax, Wo1, bo12, Wo2, bo22):
    return pl.pallas_call(
        _pool_body,
        grid=(NB,),
        in_specs=[
            pl.BlockSpec((BN, H), lambda i: (i, 0)),
            pl.BlockSpec((BN, 1), lambda i: (i, 0)),
            pl.BlockSpec((1, 1), lambda i: (0, 0)),
            pl.BlockSpec((H, 2 * H), lambda i: (0, 0)),
            pl.BlockSpec((1, 2 * H), lambda i: (0, 0)),
            pl.BlockSpec((2 * H, 2 * H), lambda i: (0, 0)),
            pl.BlockSpec((1, 2 * H), lambda i: (0, 0)),
        ],
        out_specs=pl.BlockSpec((1, 2 * H), lambda i: (0, 0)),
        out_shape=jax.ShapeDtypeStruct((1, 2 * H), jnp.float32),
        scratch_shapes=[pltpu.VMEM((8, 128), jnp.float32)],
    )(h, gate, gmax, Wo1, bo12, Wo2, bo22)


def kernel(atom, lap_pe, bond, edge_index, Wn, bnode, We, be, W1, b1, W2, b2,
           W3, b3, g1, bt1, g2, bt2, g3, bt3, Wg, bg, Wo1, bo1, Wo2, bo2):
    pad = EPAD - E
    src2 = jnp.concatenate([edge_index[0],
                            jnp.zeros((pad,), jnp.int32)]).reshape(NCHK, C)
    dst2 = jnp.concatenate([edge_index[1],
                            jnp.zeros((pad,), jnp.int32)]).reshape(NCHK, C)

    ewT = _enc_edges(bond, We, be.reshape(-1, 1))
    ew2 = jnp.concatenate([ewT.reshape(-1),
                           jnp.zeros((pad,), jnp.float32)]).reshape(NCHK, C)

    degS, degD = _deg(src2, dst2)
    x = _enc_nodes(atom, lap_pe, Wn[:44], Wn[44:], bnode.reshape(1, -1), degS)

    h = None
    x_cur = x
    for li, (W, b, g, bt) in enumerate(
            [(W1, b1, g1, bt1), (W2, b2, g2, bt2), (W3, b3, g3, bt3)]):
        aggA, aggB = _agg(x_cur, src2, dst2, ew2)
        y, st = _mm(aggA, aggB, degD, W, b.reshape(1, -1))
        if li < 2:
            x_cur = _bn_scale(y, st, g.reshape(1, -1), bt.reshape(1, -1), degS)
        else:
            h = _bn_h(y, st, g.reshape(1, -1), bt.reshape(1, -1))

    gate, gmax = _gate(h, Wg.reshape(1, -1), bg.reshape(1, 1))
    return _pool(h, gate, gmax, Wo1, bo1.reshape(1, -1), Wo2,
                 bo2.reshape(1, -1))


# speedup vs baseline: 1.3416x; 1.3416x over previous
import jax
import jax.numpy as jnp
from jax import lax
from jax.experimental import pallas as pl
from jax.experimental.pallas import tpu as pltpu
from jax.experimental.pallas import tpu_sc as plsc

N = 50000
E = 800000
H = 64
HH = 32
C = 128
NSUB = 16
EPT = 50176
EPAD = EPT * NSUB
NPAD = 50048
NPT = NPAD // NSUB
CPT = EPT // C
SCH = 1
NSC = CPT // SCH
ROWS = SCH * C
NCHK = EPAD // C
ZROWS = 391
BN = 2000
NB = N // BN
BE = 16000
PADC = float(EPAD - E)

_mesh = plsc.VectorSubcoreMesh(core_axis_name="c", subcore_axis_name="s")

_GDN = lax.GatherDimensionNumbers(offset_dims=(), collapsed_slice_dims=(0,),
                                  start_index_map=(0,))


def _vgather16(v, idx16):
    return lax.gather(v, idx16[:, None], _GDN, (1,),
                      indices_are_sorted=False, unique_indices=False,
                      mode=lax.GatherScatterMode.PROMISE_IN_BOUNDS)


def _deg_body(src2, dst2, degS, degD, ib0, ib1, ib2, ib3, onesv, zbuf, acc,
              is0, is1, is2, is3, ss0, ss1):
    c = lax.axis_index("c")
    s = lax.axis_index("s")
    ibufs = [ib0, ib1, ib2, ib3]
    isems = [is0, is1, is2, is3]
    ssems = [ss0, ss1]
    ones16 = jnp.ones((16,), jnp.float32)
    zero16 = jnp.zeros((16,), jnp.float32)

    def fill_ones(r, _):
        onesv[r, pl.ds(0, 16)] = ones16
        return _
    lax.fori_loop(0, C, fill_ones, 0)

    def fill_zero(r, _):
        zbuf[r, pl.ds(0, 16)] = zero16
        return _
    lax.fori_loop(0, ZROWS, fill_zero, 0)

    def zero_acc(i, _):
        pltpu.sync_copy(zbuf, acc.at[pl.ds(s * NPT + i * ZROWS, ZROWS)])
        return _
    lax.fori_loop(0, NPT // ZROWS, zero_acc, 0)
    plsc.subcore_barrier()

    base = s * CPT

    def src_rows(t):
        return src2.at[pl.ds(base + t * SCH, SCH)]

    def dst_rows(t):
        return dst2.at[pl.ds(base + t * SCH, SCH)]

    def fire_idx(t, b):
        @pl.when(c == 0)
        def _():
            pltpu.async_copy(src_rows(t), ibufs[b], isems[b])

        @pl.when(c == 1)
        def _():
            pltpu.async_copy(dst_rows(t), ibufs[b], isems[b])

    def wait_idx(t, b):
        @pl.when(c == 0)
        def _():
            pltpu.make_async_copy(src_rows(t), ibufs[b], isems[b]).wait()

        @pl.when(c == 1)
        def _():
            pltpu.make_async_copy(dst_rows(t), ibufs[b], isems[b]).wait()

    @pl.when(c == 0)
    def _():
        pltpu.sync_copy(src_rows(0), ib0)

    @pl.when(c == 1)
    def _():
        pltpu.sync_copy(dst_rows(0), ib0)

    fire_idx(1, 1)

    def step(i, carry):
        for P in range(4):
            t = 4 * i + P
            ib, ise, sse = ibufs[P], isems[P], ssems[P & 1]

            @pl.when(t > 0)
            def _():
                wait_idx(t, P)

            @pl.when(t >= 2)
            def _():
                for _k in range(SCH):
                    pltpu.make_async_copy(
                        onesv, acc.at[pl.ds(0, C)], sse).wait()

            for k in range(SCH):
                pltpu.async_copy(onesv, acc.at[ib.at[k]], sse, add=True)

            @pl.when(t < NSC - 2)
            def _():
                fire_idx(t + 2, (P + 2) % 4)
        return carry
    lax.fori_loop(0, NSC // 4, step, 0)

    for par in range(2):
        for _k in range(SCH):
            pltpu.make_async_copy(onesv, acc.at[pl.ds(0, C)],
                                  ssems[par]).wait()
    plsc.subcore_barrier()

    @pl.when(c == 0)
    def _():
        pltpu.sync_copy(acc.at[pl.ds(s * NPT, NPT)],
                        degS.at[pl.ds(s * NPT, NPT)])

    @pl.when(c == 1)
    def _():
        pltpu.sync_copy(acc.at[pl.ds(s * NPT, NPT)],
                        degD.at[pl.ds(s * NPT, NPT)])


@jax.jit
def _deg(src2, dst2):
    return pl.kernel(
        _deg_body,
        out_type=[jax.ShapeDtypeStruct((NPAD, 16), jnp.float32),
                  jax.ShapeDtypeStruct((NPAD, 16), jnp.float32)],
        mesh=_mesh,
        scratch_types=(
            [pltpu.VMEM((SCH, C), jnp.int32)] * 4
            + [pltpu.VMEM((C, 16), jnp.float32),
               pltpu.VMEM((ZROWS, 16), jnp.float32),
               pltpu.VMEM_SHARED((NPAD, 16), jnp.float32)]
            + [pltpu.SemaphoreType.DMA] * 6
        ),
        compiler_params=pltpu.CompilerParams(use_tc_tiling_on_sc=False),
    )(src2, dst2)


def _agg_body(x, src2, dst2, ew2, outA, outB,
              sb0, sb1, sb2, sb3, sb4, sb5, sb6, sb7,
              db0, db1, db2, db3, db4, db5, db6, db7,
              eb0, eb1, eb2, eb3, eb4, eb5, eb6, eb7,
              rows0, rows1, rows2, rows3, wrows0, wrows1, acc,
              is0, is1, is2, is3, is4, is5, is6, is7,
              gs0, gs1, gs2, gs3, ss0, ss1):
    c = lax.axis_index("c")
    s = lax.axis_index("s")
    sbufs = [sb0, sb1, sb2, sb3, sb4, sb5, sb6, sb7]
    dbufs = [db0, db1, db2, db3, db4, db5, db6, db7]
    ebufs = [eb0, eb1, eb2, eb3, eb4, eb5, eb6, eb7]
    rowsb = [rows0, rows1, rows2, rows3]
    wrowsb = [wrows0, wrows1]
    isems = [is0, is1, is2, is3, is4, is5, is6, is7]
    gsems = [gs0, gs1, gs2, gs3]
    ssems = [ss0, ss1]
    zero16 = jnp.zeros((16,), jnp.float32)

    def fill_zero(r, _):
        wrows0[r, pl.ds(0, 16)] = zero16
        wrows0[r, pl.ds(16, 16)] = zero16
        return _
    lax.fori_loop(0, ROWS, fill_zero, 0)

    def zero_acc(i, _):
        pltpu.sync_copy(wrows0, acc.at[pl.ds(s * NPT + i * ROWS, ROWS)])
        return _
    lax.fori_loop(0, 24, zero_acc, 0)
    pltpu.sync_copy(wrows0.at[pl.ds(0, 56)],
                    acc.at[pl.ds(s * NPT + 24 * ROWS, 56)])
    plsc.subcore_barrier()

    base = s * CPT
    off16 = jnp.zeros((16,), jnp.int32) + c * N

    def add_off(b):
        sb = sbufs[b]
        for g in range(8):
            sb[0, pl.ds(g * 16, 16)] = sb[0, pl.ds(g * 16, 16)] + off16

    def s_rows(t):
        return src2.at[pl.ds(base + t, 1)]

    def d_rows(t):
        return dst2.at[pl.ds(base + t, 1)]

    def e_rows(t):
        return ew2.at[pl.ds(base + t, 1)]

    def fire_idx(t, b):
        pltpu.async_copy(s_rows(t), sbufs[b], isems[b])
        pltpu.async_copy(d_rows(t), dbufs[b], isems[b])
        pltpu.async_copy(e_rows(t), ebufs[b], isems[b])

    def wait_idx(t, b):
        pltpu.make_async_copy(s_rows(t), sbufs[b], isems[b]).wait()
        pltpu.make_async_copy(d_rows(t), dbufs[b], isems[b]).wait()
        pltpu.make_async_copy(e_rows(t), ebufs[b], isems[b]).wait()

    pltpu.sync_copy(s_rows(0), sb0)
    pltpu.sync_copy(d_rows(0), db0)
    pltpu.sync_copy(e_rows(0), eb0)
    pltpu.sync_copy(s_rows(1), sb1)
    pltpu.sync_copy(d_rows(1), db1)
    pltpu.sync_copy(e_rows(1), eb1)
    fire_idx(2, 2)
    fire_idx(3, 3)
    add_off(0)
    add_off(1)
    pltpu.async_copy(x.at[sb0.at[0]], rows0, gs0)
    pltpu.async_copy(x.at[sb1.at[0]], rows1, gs1)

    def step(i, carry):
        for P in range(8):
            t = 8 * i + P
            P2 = (P + 2) % 8
            P4 = (P + 4) % 8
            r4 = P % 4
            g2 = (P + 2) % 4
            par = P & 1

            @pl.when(t < NSC - 2)
            def _():
                wait_idx(t + 2, P2)
                add_off(P2)
                pltpu.async_copy(x.at[sbufs[P2].at[0]], rowsb[g2],
                                 gsems[g2])

            pltpu.make_async_copy(x.at[pl.ds(0, C)], rowsb[r4],
                                  gsems[r4]).wait()

            @pl.when(t >= 2)
            def _():
                pltpu.make_async_copy(wrowsb[par], acc.at[pl.ds(0, C)],
                                      ssems[par]).wait()

            rb = rowsb[r4]
            wb = wrowsb[par]
            eb = ebufs[P]

            def wgrp(g, _2):
                w16 = eb[0, pl.ds(g * 16, 16)]
                rbase = g * 16
                for j in range(16):
                    r = rbase + j
                    w = _vgather16(w16, jnp.full((16,), j, jnp.int32))
                    wb[r, pl.ds(0, 16)] = rb[r, pl.ds(0, 16)] * w
                    wb[r, pl.ds(16, 16)] = rb[r, pl.ds(16, 16)] * w
                return _2
            lax.fori_loop(0, 8, wgrp, 0)

            pltpu.async_copy(wb, acc.at[dbufs[P].at[0]], ssems[par],
                             add=True)

            @pl.when(t < NSC - 4)
            def _():
                fire_idx(t + 4, P4)
        return carry
    lax.fori_loop(0, NSC // 8, step, 0)

    for par in range(2):
        pltpu.make_async_copy(wrowsb[par], acc.at[pl.ds(0, C)],
                              ssems[par]).wait()
    plsc.subcore_barrier()

    @pl.when(c == 0)
    def _():
        pltpu.sync_copy(acc.at[pl.ds(s * NPT, NPT)],
                        outA.at[pl.ds(s * NPT, NPT)])

    @pl.when(c == 1)
    def _():
        pltpu.sync_copy(acc.at[pl.ds(s * NPT, NPT)],
                        outB.at[pl.ds(s * NPT, NPT)])


@jax.jit
def _agg(x, src2, dst2, ew2):
    return pl.kernel(
        _agg_body,
        out_type=[jax.ShapeDtypeStruct((NPAD, HH), jnp.float32),
                  jax.ShapeDtypeStruct((NPAD, HH), jnp.float32)],
        mesh=_mesh,
        scratch_types=(
            [pltpu.VMEM((1, C), jnp.int32)] * 16
            + [pltpu.VMEM((1, C), jnp.float32)] * 8
            + [pltpu.VMEM((ROWS, HH), jnp.float32)] * 6
            + [pltpu.VMEM_SHARED((NPAD, HH), jnp.float32)]
            + [pltpu.SemaphoreType.DMA] * 14
        ),
        compiler_params=pltpu.CompilerParams(use_tc_tiling_on_sc=False),
    )(x, src2, dst2, ew2)


def _relu(v):
    return jnp.maximum(v, 0.0)


def _enc_edges_body(bond_ref, we_ref, be_ref, ew_ref):
    zt = lax.dot_general(we_ref[...], bond_ref[...],
                         (((0,), (1,)), ((), ())),
                         preferred_element_type=jnp.float32)
    e = _relu(zt + be_ref[...])
    ew_ref[...] = jnp.mean(e, axis=0, keepdims=True)


@jax.jit
def _enc_edges(bond, We, beT):
    return pl.pallas_call(
        _enc_edges_body,
        grid=(E // BE,),
        in_specs=[
            pl.BlockSpec((BE, 10), lambda i: (i, 0)),
            pl.BlockSpec((10, H), lambda i: (0, 0)),
            pl.BlockSpec((H, 1), lambda i: (0, 0)),
        ],
        out_specs=pl.BlockSpec((1, BE), lambda i: (0, i)),
        out_shape=jax.ShapeDtypeStruct((1, E), jnp.float32),
    )(bond, We, beT)


def _norm_from_deg(deg_col, blk_i):
    rowz = lax.broadcasted_iota(jnp.int32, (BN, 1), 0) == 0
    fix = jnp.where(rowz & (blk_i == 0), PADC, 0.0)
    d = jnp.maximum(deg_col - fix, 1.0)
    return lax.rsqrt(d)


def _enc_nodes_body(atom_ref, lap_ref, wa_ref, wb_ref, b_ref, deg_ref, x_ref):
    i = pl.program_id(0)
    c = pl.program_id(1)
    h = _relu(jnp.dot(atom_ref[...], wa_ref[...],
                      preferred_element_type=jnp.float32)
              + jnp.dot(lap_ref[...], wb_ref[...],
                        preferred_element_type=jnp.float32)
              + b_ref[...])
    x = h * _norm_from_deg(deg_ref[:, 0:1], i)
    x_ref[...] = jnp.where(c == 0, x[:, :HH], x[:, HH:])


@jax.jit
def _enc_nodes(atom, lap_pe, Wa, Wb, b2, degS):
    return pl.pallas_call(
        _enc_nodes_body,
        grid=(NB, 2),
        in_specs=[
            pl.BlockSpec((BN, 44), lambda i, c: (i, 0)),
            pl.BlockSpec((BN, 8), lambda i, c: (i, 0)),
            pl.BlockSpec((44, H), lambda i, c: (0, 0)),
            pl.BlockSpec((8, H), lambda i, c: (0, 0)),
            pl.BlockSpec((1, H), lambda i, c: (0, 0)),
            pl.BlockSpec((BN, 16), lambda i, c: (i, 0)),
        ],
        out_specs=pl.BlockSpec((BN, HH), lambda i, c: (c * NB + i, 0)),
        out_shape=jax.ShapeDtypeStruct((2 * N, HH), jnp.float32),
    )(atom, lap_pe, Wa, Wb, b2, degS)


def _mm_body(aggA_ref, aggB_ref, deg_ref, w_ref, b_ref, y_ref, st_ref):
    i = pl.program_id(0)
    z = jnp.concatenate([aggA_ref[...], aggB_ref[...]], axis=1)
    z = z * _norm_from_deg(deg_ref[:, 0:1], i)
    y = jnp.dot(z, w_ref[...], preferred_element_type=jnp.float32) + b_ref[...]
    y_ref[...] = y
    s1 = jnp.sum(y, axis=0)
    s2 = jnp.sum(y * y, axis=0)
    z64 = jnp.zeros((1, H), jnp.float32)
    st = jnp.concatenate(
        [jnp.concatenate([s1.reshape(1, H), z64], axis=1),
         jnp.concatenate([s2.reshape(1, H), z64], axis=1),
         jnp.zeros((6, 128), jnp.float32)], axis=0)

    @pl.when(i == 0)
    def _():
        st_ref[...] = st

    @pl.when(i > 0)
    def _():
        st_ref[...] = st_ref[...] + st


@jax.jit
def _mm(aggA, aggB, degD, W, b2):
    return pl.pallas_call(
        _mm_body,
        grid=(NB,),
        in_specs=[
            pl.BlockSpec((BN, HH), lambda i: (i, 0)),
            pl.BlockSpec((BN, HH), lambda i: (i, 0)),
            pl.BlockSpec((BN, 16), lambda i: (i, 0)),
            pl.BlockSpec((H, H), lambda i: (0, 0)),
            pl.BlockSpec((1, H), lambda i: (0, 0)),
        ],
        out_specs=[
            pl.BlockSpec((BN, H), lambda i: (i, 0)),
            pl.BlockSpec((8, 128), lambda i: (0, 0)),
        ],
        out_shape=[
            jax.ShapeDtypeStruct((N, H), jnp.float32),
            jax.ShapeDtypeStruct((8, 128), jnp.float32),
        ],
    )(aggA, aggB, degD, W, b2)


def _bn(y, st_ref, g_ref, bt_ref):
    mu = st_ref[0:1, 0:H] / N
    var = st_ref[1:2, 0:H] / N - mu * mu
    return _relu(g_ref[...] * (y - mu) * lax.rsqrt(var + 1e-5) + bt_ref[...])


def _bn_scale_body(y_ref, st_ref, g_ref, bt_ref, deg_ref, x_ref):
    i = pl.program_id(0)
    c = pl.program_id(1)
    h = _bn(y_ref[...], st_ref, g_ref, bt_ref)
    x = h * _norm_from_deg(deg_ref[:, 0:1], i)
    x_ref[...] = jnp.where(c == 0, x[:, :HH], x[:, HH:])


@jax.jit
def _bn_scale(y, st, g2, bt2, degS):
    return pl.pallas_call(
        _bn_scale_body,
        grid=(NB, 2),
        in_specs=[
            pl.BlockSpec((BN, H), lambda i, c: (i, 0)),
            pl.BlockSpec((8, 128), lambda i, c: (0, 0)),
            pl.BlockSpec((1, H), lambda i, c: (0, 0)),
            pl.BlockSpec((1, H), lambda i, c: (0, 0)),
            pl.BlockSpec((BN, 16), lambda i, c: (i, 0)),
        ],
        out_specs=pl.BlockSpec((BN, HH), lambda i, c: (c * NB + i, 0)),
        out_shape=jax.ShapeDtypeStruct((2 * N, HH), jnp.float32),
    )(y, st, g2, bt2, degS)


def _bn_h_body(y_ref, st_ref, g_ref, bt_ref, h_ref):
    h_ref[...] = _bn(y_ref[...], st_ref, g_ref, bt_ref)


@jax.jit
def _bn_h(y, st, g2, bt2):
    return pl.pallas_call(
        _bn_h_body,
        grid=(NB,),
        in_specs=[
            pl.BlockSpec((BN, H), lambda i: (i, 0)),
            pl.BlockSpec((8, 128), lambda i: (0, 0)),
            pl.BlockSpec((1, H), lambda i: (0, 0)),
            pl.BlockSpec((1, H), lambda i: (0, 0)),
        ],
        out_specs=pl.BlockSpec((BN, H), lambda i: (i, 0)),
        out_shape=jax.ShapeDtypeStruct((N, H), jnp.float32),
    )(y, st, g2, bt2)


def _gate_body(h_ref, wg_ref, bg_ref, gate_ref, gmax_ref):
    i = pl.program_id(0)
    gate = jnp.sum(h_ref[...] * wg_ref[...], axis=1, keepdims=True) + bg_ref[...]
    gate_ref[...] = gate
    m = jnp.max(gate).reshape(1, 1)

    @pl.when(i == 0)
    def _():
        gmax_ref[...] = m

    @pl.when(i > 0)
    def _():
        gmax_ref[...] = jnp.maximum(gmax_ref[...], m)


@jax.jit
def _gate(h, wg2, bg2):
    return pl.pallas_call(
        _gate_body,
        grid=(NB,),
        in_specs=[
            pl.BlockSpec((BN, H), lambda i: (i, 0)),
            pl.BlockSpec((1, H), lambda i: (0, 0)),
            pl.BlockSpec((1, 1), lambda i: (0, 0)),
        ],
        out_specs=[
            pl.BlockSpec((BN, 1), lambda i: (i, 0)),
            pl.BlockSpec((1, 1), lambda i: (0, 0)),
        ],
        out_shape=[
            jax.ShapeDtypeStruct((N, 1), jnp.float32),
            jax.ShapeDtypeStruct((1, 1), jnp.float32),
        ],
    )(h, wg2, bg2)


def _pool_body(h_ref, gate_ref, gmax_ref, wo1_ref, bo1_ref, wo2_ref, bo2_ref,
               out_ref, acc_ref):
    i = pl.program_id(0)
    a = jnp.exp(gate_ref[...] - gmax_ref[0, 0])
    s = jnp.sum(a)
    r = jnp.sum(a * h_ref[...], axis=0)
    cur = jnp.concatenate(
        [jnp.concatenate([r.reshape(1, H), jnp.zeros((1, H), jnp.float32)],
                         axis=1),
         jnp.concatenate([s.reshape(1, 1), jnp.zeros((1, 127), jnp.float32)],
                         axis=1),
         jnp.zeros((6, 128), jnp.float32)], axis=0)

    @pl.when(i == 0)
    def _():
        acc_ref[...] = cur

    @pl.when(i > 0)
    def _():
        acc_ref[...] = acc_ref[...] + cur

    @pl.when(i == NB - 1)
    def _():
        rd = acc_ref[0:1, 0:H] / acc_ref[1, 0]
        o = _relu(jnp.dot(rd, wo1_ref[...],
                          preferred_element_type=jnp.float32) + bo1_ref[...])
        out_ref[...] = jnp.dot(o, wo2_ref[...],
                               preferred_element_type=jnp.float32) + bo2_ref[...]


@jax.jit
def _pool(h, gate, gmax, Wo1, bo12, Wo2, bo22):
    return pl.pallas_call(
        _pool_body,
        grid=(NB,),
        in_specs=[
            pl.BlockSpec((BN, H), lambda i: (i, 0)),
            pl.BlockSpec((BN, 1), lambda i: (i, 0)),
            pl.BlockSpec((1, 1), lambda i: (0, 0)),
            pl.BlockSpec((H, 2 * H), lambda i: (0, 0)),
            pl.BlockSpec((1, 2 * H), lambda i: (0, 0)),
            pl.BlockSpec((2 * H, 2 * H), lambda i: (0, 0)),
            pl.BlockSpec((1, 2 * H), lambda i: (0, 0)),
        ],
        out_specs=pl.BlockSpec((1, 2 * H), lambda i: (0, 0)),
        out_shape=jax.ShapeDtypeStruct((1, 2 * H), jnp.float32),
        scratch_shapes=[pltpu.VMEM((8, 128), jnp.float32)],
    )(h, gate, gmax, Wo1, bo12, Wo2, bo22)


def kernel(atom, lap_pe, bond, edge_index, Wn, bnode, We, be, W1, b1, W2, b2,
           W3, b3, g1, bt1, g2, bt2, g3, bt3, Wg, bg, Wo1, bo1, Wo2, bo2):
    pad = EPAD - E
    src2 = jnp.concatenate([edge_index[0],
                            jnp.zeros((pad,), jnp.int32)]).reshape(NCHK, C)
    dst2 = jnp.concatenate([edge_index[1],
                            jnp.zeros((pad,), jnp.int32)]).reshape(NCHK, C)

    ewT = _enc_edges(bond, We, be.reshape(-1, 1))
    ew2 = jnp.concatenate([ewT.reshape(-1),
                           jnp.zeros((pad,), jnp.float32)]).reshape(NCHK, C)

    degS, degD = _deg(src2, dst2)
    x = _enc_nodes(atom, lap_pe, Wn[:44], Wn[44:], bnode.reshape(1, -1), degS)

    h = None
    x_cur = x
    for li, (W, b, g, bt) in enumerate(
            [(W1, b1, g1, bt1), (W2, b2, g2, bt2), (W3, b3, g3, bt3)]):
        aggA, aggB = _agg(x_cur, src2, dst2, ew2)
        y, st = _mm(aggA, aggB, degD, W, b.reshape(1, -1))
        if li < 2:
            x_cur = _bn_scale(y, st, g.reshape(1, -1), bt.reshape(1, -1), degS)
        else:
            h = _bn_h(y, st, g.reshape(1, -1), bt.reshape(1, -1))

    gate, gmax = _gate(h, Wg.reshape(1, -1), bg.reshape(1, 1))
    return _pool(h, gate, gmax, Wo1, bo1.reshape(1, -1), Wo2,
                 bo2.reshape(1, -1))
